# baseline (speedup 1.0000x reference)
import functools
import jax
import jax.numpy as jnp
from jax import lax
from jax.experimental import pallas as pl
from jax.experimental.pallas import tpu as pltpu
from jax.experimental.pallas import tpu_sc as plsc

N_NODES = 100000
N_EDGES = 1600000
IN_CH = 128
HID = 32

GROUP = 12800
NPASS = 4
E_PAD = 1638400
PAD_N = E_PAD - N_EDGES
WIN = 2048
NDUMMY = 512
AGG_ROWS = 13312
CNT_ROWS = 100352
ROW_BLK = 2000

_MESH = plsc.VectorSubcoreMesh(core_axis_name="c", subcore_axis_name="s")


def _prep(dst_p):
    @functools.partial(
        pl.kernel,
        out_type=(
            jax.ShapeDtypeStruct((CNT_ROWS,), jnp.float32),
            jax.ShapeDtypeStruct((CNT_ROWS,), jnp.float32),
        ),
        mesh=_MESH,
        scratch_types=[
            pltpu.VMEM((WIN,), jnp.int32),
            pltpu.VMEM((WIN,), jnp.float32),
            pltpu.VMEM((6272,), jnp.float32),
            pltpu.VMEM_SHARED((CNT_ROWS,), jnp.float32),
        ],
        compiler_params=pltpu.CompilerParams(use_tc_tiling_on_sc=False, needs_layout_passes=False),
    )
    def k(dst_hbm, cnt0_hbm, cnt1_hbm, dst_v, ones_v, zc_v, cnt_sp):
        c = lax.axis_index("c")
        s = lax.axis_index("s")
        wid = c * 16 + s

        def fill(i, _):
            ones_v[pl.ds(i * 16, 16)] = jnp.full((16,), 1.0, jnp.float32)
            return 0
        lax.fori_loop(0, WIN // 16, fill, 0)

        def zz(i, _):
            zc_v[pl.ds(i * 16, 16)] = jnp.zeros((16,), jnp.float32)
            return 0
        lax.fori_loop(0, 6272 // 16, zz, 0)
        pltpu.sync_copy(zc_v, cnt_sp.at[pl.ds(s * 6272, 6272)])
        plsc.subcore_barrier()

        def win(w, _):
            base = wid * (E_PAD // 32) + w * WIN
            pltpu.sync_copy(dst_hbm.at[pl.ds(base, WIN)], dst_v)
            pltpu.sync_copy(ones_v, cnt_sp.at[dst_v], add=True)
            return 0
        lax.fori_loop(0, (E_PAD // 32) // WIN, win, 0)
        plsc.subcore_barrier()

        pltpu.sync_copy(cnt_sp.at[pl.ds(s * 6272, 6272)], zc_v)

        @pl.when(c == 0)
        def _():
            pltpu.sync_copy(zc_v, cnt0_hbm.at[pl.ds(s * 6272, 6272)])

        @pl.when(c == 1)
        def _():
            pltpu.sync_copy(zc_v, cnt1_hbm.at[pl.ds(s * 6272, 6272)])

    return k(dst_p)


def _conv(h, src_p, dst_p):
    @functools.partial(
        pl.kernel,
        out_type=jax.ShapeDtypeStruct((8 * GROUP, HID), jnp.float32),
        mesh=_MESH,
        scratch_types=[
            pltpu.VMEM((WIN,), jnp.int32),
            pltpu.VMEM((WIN,), jnp.int32),
            pltpu.VMEM((WIN,), jnp.int32),
            pltpu.VMEM((WIN,), jnp.int32),
            pltpu.VMEM((WIN, HID), jnp.float32),
            pltpu.VMEM((832, HID), jnp.float32),
            pltpu.VMEM_SHARED((AGG_ROWS, HID), jnp.float32),
            pltpu.SemaphoreType.DMA,
        ],
        compiler_params=pltpu.CompilerParams(use_tc_tiling_on_sc=False, needs_layout_passes=False),
    )
    def k(h_hbm, src_hbm, dst_hbm, agg_hbm,
          src_v, dst_v, psrc_v, pldst_v, rows_v, out_v, agg_sp, sem):
        c = lax.axis_index("c")
        s = lax.axis_index("s")
        lanes = lax.iota(jnp.int32, 16)
        pad_src = jnp.bitwise_and(lanes * 31, NDUMMY - 1)
        pad_ldst = GROUP + pad_src

        def flush():
            pltpu.async_copy(h_hbm.at[psrc_v], rows_v, sem).wait()
            pltpu.sync_copy(rows_v, agg_sp.at[pldst_v], add=True)

        def pad_to(cnt, hi):
            m = lanes < hi - cnt
            idx = jnp.minimum(cnt + lanes, WIN - 1)
            plsc.store_scatter(psrc_v, [idx], pad_src, mask=m)
            plsc.store_scatter(pldst_v, [idx], pad_ldst, mask=m)

        for p in range(NPASS):
            qbase = (2 * p + c) * GROUP

            def z(i, _):
                out_v[i, pl.ds(0, 16)] = jnp.zeros((16,), jnp.float32)
                out_v[i, pl.ds(16, 16)] = jnp.zeros((16,), jnp.float32)
                return 0
            lax.fori_loop(0, 832, z, 0)
            pltpu.sync_copy(out_v, agg_sp.at[pl.ds(s * 832, 832)])
            plsc.subcore_barrier()

            def win(w, cnt):
                base = s * (E_PAD // 16) + w * WIN
                pltpu.sync_copy(src_hbm.at[pl.ds(base, WIN)], src_v)
                pltpu.sync_copy(dst_hbm.at[pl.ds(base, WIN)], dst_v)

                def grp(j, cnt):
                    d = dst_v[pl.ds(j * 16, 16)] - qbase
                    ok = jnp.logical_and(d >= 0, d < GROUP)
                    gc = jnp.sum(jnp.where(ok, 1, 0))
                    full = cnt > WIN - 16

                    @pl.when(full)
                    def _():
                        pad_to(cnt, jnp.int32(WIN))
                        flush()

                    cnt = jnp.where(full, 0, cnt)
                    sv = src_v[pl.ds(j * 16, 16)]
                    plsc.store_compressed(psrc_v.at[pl.ds(cnt, 16)], sv, mask=ok)
                    plsc.store_compressed(pldst_v.at[pl.ds(cnt, 16)], d, mask=ok)
                    return cnt + gc
                return lax.fori_loop(0, WIN // 16, grp, cnt)
            cnt = lax.fori_loop(0, (E_PAD // 16) // WIN, win, jnp.int32(0))

            def tail(i, cnt):
                pad_to(cnt, jnp.int32(WIN))
                return jnp.minimum(cnt + 16, WIN)
            lax.fori_loop(0, WIN // 16, tail, cnt)
            flush()
            plsc.subcore_barrier()

            ro = s * 800
            ob = (2 * p + c) * GROUP
            pltpu.sync_copy(agg_sp.at[pl.ds(ro, 800)],
                            out_v.at[pl.ds(0, 800)])
            pltpu.sync_copy(out_v.at[pl.ds(0, 800)],
                            agg_hbm.at[pl.ds(ob + ro, 800)])

            plsc.subcore_barrier()

    return k(h, src_p, dst_p)


def _mm_relu_body(x_ref, w_ref, b_ref, o_ref):
    o_ref[...] = jax.nn.relu(
        jnp.dot(x_ref[...], w_ref[...], preferred_element_type=jnp.float32)
        + b_ref[...]
    )


def _mm_relu(x, w, b):
    m, kk = x.shape
    n = w.shape[1]
    return pl.pallas_call(
        _mm_relu_body,
        grid=(m // ROW_BLK,),
        in_specs=[
            pl.BlockSpec((ROW_BLK, kk), lambda i: (i, 0)),
            pl.BlockSpec((kk, n), lambda i: (0, 0)),
            pl.BlockSpec((1, n), lambda i: (0, 0)),
        ],
        out_specs=pl.BlockSpec((ROW_BLK, n), lambda i: (i, 0)),
        out_shape=jax.ShapeDtypeStruct((m, n), jnp.float32),
    )(x, w, b.reshape(1, n))


def _combine_body(agg_ref, c0_ref, c1_ref, h_ref, wl_ref, wr_ref, b_ref,
                  o_ref):
    cnt = jnp.maximum(c0_ref[...] + c1_ref[...], 1.0)
    o_ref[...] = jax.nn.relu(
        jnp.dot(agg_ref[...] / cnt, wl_ref[...],
                preferred_element_type=jnp.float32)
        + jnp.dot(h_ref[...], wr_ref[...], preferred_element_type=jnp.float32)
        + b_ref[...]
    )


def _combine2_body(agg_ref, c0_ref, c1_ref, h_ref, wl_ref, wr_ref, b_ref,
                   w2_ref, b2_ref, o_ref):
    cnt = jnp.maximum(c0_ref[...] + c1_ref[...], 1.0)
    t = jax.nn.relu(
        jnp.dot(agg_ref[...] / cnt, wl_ref[...],
                preferred_element_type=jnp.float32)
        + jnp.dot(h_ref[...], wr_ref[...], preferred_element_type=jnp.float32)
        + b_ref[...]
    )
    o_ref[...] = jax.nn.relu(
        jnp.dot(t, w2_ref[...], preferred_element_type=jnp.float32)
        + b2_ref[...]
    )


def _combine(agg, c0, c1, h, wl, wr, b, w2=None, b2=None):
    m, n = h.shape
    fused = w2 is not None
    body = _combine2_body if fused else _combine_body
    specs = [
        pl.BlockSpec((ROW_BLK, n), lambda i: (i, 0)),
        pl.BlockSpec((ROW_BLK, 1), lambda i: (i, 0)),
        pl.BlockSpec((ROW_BLK, 1), lambda i: (i, 0)),
        pl.BlockSpec((ROW_BLK, n), lambda i: (i, 0)),
        pl.BlockSpec((n, n), lambda i: (0, 0)),
        pl.BlockSpec((n, n), lambda i: (0, 0)),
        pl.BlockSpec((1, n), lambda i: (0, 0)),
    ]
    args = [agg, c0, c1, h, wl, wr, b.reshape(1, n)]
    if fused:
        specs += [
            pl.BlockSpec((n, n), lambda i: (0, 0)),
            pl.BlockSpec((1, n), lambda i: (0, 0)),
        ]
        args += [w2, b2.reshape(1, n)]
    return pl.pallas_call(
        body,
        grid=(m // ROW_BLK,),
        in_specs=specs,
        out_specs=pl.BlockSpec((ROW_BLK, n), lambda i: (i, 0)),
        out_shape=jax.ShapeDtypeStruct((m, n), jnp.float32),
    )(*args)


def kernel(x, edge_index, W1, b1, W2, b2, Wl1, Wr1, bc1, Wl2, Wr2, bc2):
    src = edge_index[0].astype(jnp.int32)
    dst = edge_index[1].astype(jnp.int32)

    src_p = jnp.concatenate([src, jnp.zeros((PAD_N,), jnp.int32)])
    dst_p = jnp.concatenate(
        [dst, N_NODES + (jnp.arange(PAD_N, dtype=jnp.int32) & 255)])

    cnt0, cnt1 = _prep(dst_p)
    c0 = cnt0[:N_NODES].reshape(N_NODES, 1)
    c1 = cnt1[:N_NODES].reshape(N_NODES, 1)

    h0 = _mm_relu(x, W1, b1)
    agg1 = _conv(h0, src_p, dst_p)[:N_NODES]
    h2 = _combine(agg1, c0, c1, h0, Wl1, Wr1, bc1, W2, b2)
    agg2 = _conv(h2, src_p, dst_p)[:N_NODES]
    return _combine(agg2, c0, c1, h2, Wl2, Wr2, bc2)


# speedup vs baseline: 1.4022x; 1.4022x over previous
import functools
import jax
import jax.numpy as jnp
from jax import lax
from jax.experimental import pallas as pl
from jax.experimental.pallas import tpu as pltpu
from jax.experimental.pallas import tpu_sc as plsc

N_NODES = 100000
N_EDGES = 1600000
IN_CH = 128
HID = 32

GROUP = 12800
NPASS = 4
E_PAD = 1638400
PAD_N = E_PAD - N_EDGES
WIN = 2048
NDUMMY = 512
AGG_ROWS = 13312
CNT_ROWS = 102912
ROW_BLK = 2000

_MESH = plsc.VectorSubcoreMesh(core_axis_name="c", subcore_axis_name="s")


def _prep(dst_p):
    @functools.partial(
        pl.kernel,
        out_type=(
            jax.ShapeDtypeStruct((CNT_ROWS,), jnp.float32),
            jax.ShapeDtypeStruct((CNT_ROWS,), jnp.float32),
        ),
        mesh=_MESH,
        scratch_types=[
            pltpu.VMEM((WIN,), jnp.int32),
            pltpu.VMEM((WIN,), jnp.float32),
            pltpu.VMEM((6432,), jnp.float32),
            pltpu.VMEM_SHARED((CNT_ROWS,), jnp.float32),
        ],
        compiler_params=pltpu.CompilerParams(use_tc_tiling_on_sc=False, needs_layout_passes=False),
    )
    def k(dst_hbm, cnt0_hbm, cnt1_hbm, dst_v, ones_v, zc_v, cnt_sp):
        c = lax.axis_index("c")
        s = lax.axis_index("s")
        wid = c * 16 + s

        def fill(i, _):
            ones_v[pl.ds(i * 16, 16)] = jnp.full((16,), 1.0, jnp.float32)
            return 0
        lax.fori_loop(0, WIN // 16, fill, 0)

        def zz(i, _):
            zc_v[pl.ds(i * 16, 16)] = jnp.zeros((16,), jnp.float32)
            return 0
        lax.fori_loop(0, 6432 // 16, zz, 0)
        pltpu.sync_copy(zc_v, cnt_sp.at[pl.ds(s * 6432, 6432)])
        plsc.subcore_barrier()

        def win(w, _):
            base = wid * (E_PAD // 32) + w * WIN
            pltpu.sync_copy(dst_hbm.at[pl.ds(base, WIN)], dst_v)
            pltpu.sync_copy(ones_v, cnt_sp.at[dst_v], add=True)
            return 0
        lax.fori_loop(0, (E_PAD // 32) // WIN, win, 0)
        plsc.subcore_barrier()

        pltpu.sync_copy(cnt_sp.at[pl.ds(s * 6432, 6432)], zc_v)

        @pl.when(c == 0)
        def _():
            pltpu.sync_copy(zc_v, cnt0_hbm.at[pl.ds(s * 6432, 6432)])

        @pl.when(c == 1)
        def _():
            pltpu.sync_copy(zc_v, cnt1_hbm.at[pl.ds(s * 6432, 6432)])

    return k(dst_p)


def _conv(h, src_p, dst_p):
    @functools.partial(
        pl.kernel,
        out_type=jax.ShapeDtypeStruct((8 * GROUP, HID), jnp.float32),
        mesh=_MESH,
        scratch_types=[
            pltpu.VMEM((WIN,), jnp.int32),
            pltpu.VMEM((WIN,), jnp.int32),
            pltpu.VMEM((WIN,), jnp.int32),
            pltpu.VMEM((WIN,), jnp.int32),
            pltpu.VMEM((WIN, HID), jnp.float32),
            pltpu.VMEM((832, HID), jnp.float32),
            pltpu.VMEM_SHARED((AGG_ROWS, HID), jnp.float32),
            pltpu.SemaphoreType.DMA,
        ],
        compiler_params=pltpu.CompilerParams(use_tc_tiling_on_sc=False, needs_layout_passes=False),
    )
    def k(h_hbm, src_hbm, dst_hbm, agg_hbm,
          src_v, dst_v, psrc_v, pldst_v, rows_v, out_v, agg_sp, sem):
        c = lax.axis_index("c")
        s = lax.axis_index("s")
        lanes = lax.iota(jnp.int32, 16)
        pad_src = jnp.bitwise_and(lanes * 31, NDUMMY - 1)
        pad_ldst = GROUP + pad_src

        def flush():
            pltpu.async_copy(h_hbm.at[psrc_v], rows_v, sem).wait()
            pltpu.sync_copy(rows_v, agg_sp.at[pldst_v], add=True)

        def pad_to(cnt, hi):
            m = lanes < hi - cnt
            idx = jnp.minimum(cnt + lanes, WIN - 1)
            plsc.store_scatter(psrc_v, [idx], pad_src, mask=m)
            plsc.store_scatter(pldst_v, [idx], pad_ldst, mask=m)

        for p in range(NPASS):
            qbase = (2 * p + c) * GROUP

            def z(i, _):
                out_v[i, pl.ds(0, 16)] = jnp.zeros((16,), jnp.float32)
                out_v[i, pl.ds(16, 16)] = jnp.zeros((16,), jnp.float32)
                return 0
            lax.fori_loop(0, 832, z, 0)
            pltpu.sync_copy(out_v, agg_sp.at[pl.ds(s * 832, 832)])
            plsc.subcore_barrier()

            def win(w, cnt):
                base = s * (E_PAD // 16) + w * WIN
                pltpu.sync_copy(src_hbm.at[pl.ds(base, WIN)], src_v)
                pltpu.sync_copy(dst_hbm.at[pl.ds(base, WIN)], dst_v)

                def grp(j, cnt):
                    d = dst_v[pl.ds(j * 16, 16)] - qbase
                    ok = jnp.logical_and(d >= 0, d < GROUP)
                    gc = jnp.sum(jnp.where(ok, 1, 0))
                    full = cnt > WIN - 16

                    @pl.when(full)
                    def _():
                        pad_to(cnt, jnp.int32(WIN))
                        flush()

                    cnt = jnp.where(full, 0, cnt)
                    sv = src_v[pl.ds(j * 16, 16)]
                    plsc.store_compressed(psrc_v.at[pl.ds(cnt, 16)], sv, mask=ok)
                    plsc.store_compressed(pldst_v.at[pl.ds(cnt, 16)], d, mask=ok)
                    return cnt + gc
                return lax.fori_loop(0, WIN // 16, grp, cnt)
            cnt = lax.fori_loop(0, (E_PAD // 16) // WIN, win, jnp.int32(0))

            def tail(i, cnt):
                pad_to(cnt, jnp.int32(WIN))
                return jnp.minimum(cnt + 16, WIN)
            lax.fori_loop(0, WIN // 16, tail, cnt)
            flush()
            plsc.subcore_barrier()

            ro = s * 800
            ob = (2 * p + c) * GROUP
            pltpu.sync_copy(agg_sp.at[pl.ds(ro, 800)],
                            out_v.at[pl.ds(0, 800)])
            pltpu.sync_copy(out_v.at[pl.ds(0, 800)],
                            agg_hbm.at[pl.ds(ob + ro, 800)])

            plsc.subcore_barrier()

    return k(h, src_p, dst_p)


def _mm_relu_body(x_ref, w_ref, b_ref, o_ref):
    o_ref[...] = jax.nn.relu(
        jnp.dot(x_ref[...], w_ref[...], preferred_element_type=jnp.float32)
        + b_ref[...]
    )


def _mm_relu(x, w, b):
    m, kk = x.shape
    n = w.shape[1]
    return pl.pallas_call(
        _mm_relu_body,
        grid=(m // ROW_BLK,),
        in_specs=[
            pl.BlockSpec((ROW_BLK, kk), lambda i: (i, 0)),
            pl.BlockSpec((kk, n), lambda i: (0, 0)),
            pl.BlockSpec((1, n), lambda i: (0, 0)),
        ],
        out_specs=pl.BlockSpec((ROW_BLK, n), lambda i: (i, 0)),
        out_shape=jax.ShapeDtypeStruct((m, n), jnp.float32),
    )(x, w, b.reshape(1, n))


def _combine_body(agg_ref, c0_ref, c1_ref, h_ref, wl_ref, wr_ref, b_ref,
                  o_ref):
    cnt = jnp.maximum(c0_ref[...] + c1_ref[...], 1.0)
    o_ref[...] = jax.nn.relu(
        jnp.dot(agg_ref[...] / cnt, wl_ref[...],
                preferred_element_type=jnp.float32)
        + jnp.dot(h_ref[...], wr_ref[...], preferred_element_type=jnp.float32)
        + b_ref[...]
    )


def _combine2_body(agg_ref, c0_ref, c1_ref, h_ref, wl_ref, wr_ref, b_ref,
                   w2_ref, b2_ref, o_ref):
    cnt = jnp.maximum(c0_ref[...] + c1_ref[...], 1.0)
    t = jax.nn.relu(
        jnp.dot(agg_ref[...] / cnt, wl_ref[...],
                preferred_element_type=jnp.float32)
        + jnp.dot(h_ref[...], wr_ref[...], preferred_element_type=jnp.float32)
        + b_ref[...]
    )
    o_ref[...] = jax.nn.relu(
        jnp.dot(t, w2_ref[...], preferred_element_type=jnp.float32)
        + b2_ref[...]
    )


def _combine(agg, c0, c1, h, wl, wr, b, w2=None, b2=None):
    m, n = h.shape
    fused = w2 is not None
    body = _combine2_body if fused else _combine_body
    specs = [
        pl.BlockSpec((ROW_BLK, n), lambda i: (i, 0)),
        pl.BlockSpec((ROW_BLK, 1), lambda i: (i, 0)),
        pl.BlockSpec((ROW_BLK, 1), lambda i: (i, 0)),
        pl.BlockSpec((ROW_BLK, n), lambda i: (i, 0)),
        pl.BlockSpec((n, n), lambda i: (0, 0)),
        pl.BlockSpec((n, n), lambda i: (0, 0)),
        pl.BlockSpec((1, n), lambda i: (0, 0)),
    ]
    args = [agg, c0, c1, h, wl, wr, b.reshape(1, n)]
    if fused:
        specs += [
            pl.BlockSpec((n, n), lambda i: (0, 0)),
            pl.BlockSpec((1, n), lambda i: (0, 0)),
        ]
        args += [w2, b2.reshape(1, n)]
    return pl.pallas_call(
        body,
        grid=(m // ROW_BLK,),
        in_specs=specs,
        out_specs=pl.BlockSpec((ROW_BLK, n), lambda i: (i, 0)),
        out_shape=jax.ShapeDtypeStruct((m, n), jnp.float32),
    )(*args)


def kernel(x, edge_index, W1, b1, W2, b2, Wl1, Wr1, bc1, Wl2, Wr2, bc2):
    src = edge_index[0].astype(jnp.int32)
    dst = edge_index[1].astype(jnp.int32)

    src_p = jnp.concatenate([src, jnp.zeros((PAD_N,), jnp.int32)])
    dst_p = jnp.concatenate(
        [dst, 8 * GROUP + (jnp.arange(PAD_N, dtype=jnp.int32) & 255)])

    cnt0, cnt1 = _prep(dst_p)
    c0 = cnt0[:N_NODES].reshape(N_NODES, 1)
    c1 = cnt1[:N_NODES].reshape(N_NODES, 1)

    h0 = _mm_relu(x, W1, b1)
    agg1 = _conv(h0, src_p, dst_p)[:N_NODES]
    h2 = _combine(agg1, c0, c1, h0, Wl1, Wr1, bc1, W2, b2)
    agg2 = _conv(h2, src_p, dst_p)[:N_NODES]
    return _combine(agg2, c0, c1, h2, Wl2, Wr2, bc2)


# speedup vs baseline: 1.4919x; 1.0640x over previous
import functools
import jax
import jax.numpy as jnp
from jax import lax
from jax.experimental import pallas as pl
from jax.experimental.pallas import tpu as pltpu
from jax.experimental.pallas import tpu_sc as plsc

N_NODES = 100000
N_EDGES = 1600000
IN_CH = 128
HID = 32

GROUP = 12800
NPASS = 4
E_PAD = 1638400
PAD_N = E_PAD - N_EDGES
WIN = 2048
NDUMMY = 512
AGG_ROWS = 13312
CNT_ROWS = 102912
ROW_BLK = 2000

_MESH = plsc.VectorSubcoreMesh(core_axis_name="c", subcore_axis_name="s")


def _prep(dst_p):
    @functools.partial(
        pl.kernel,
        out_type=(
            jax.ShapeDtypeStruct((CNT_ROWS,), jnp.float32),
            jax.ShapeDtypeStruct((CNT_ROWS,), jnp.float32),
        ),
        mesh=_MESH,
        scratch_types=[
            pltpu.VMEM((WIN,), jnp.int32),
            pltpu.VMEM((WIN,), jnp.float32),
            pltpu.VMEM((6432,), jnp.float32),
            pltpu.VMEM_SHARED((CNT_ROWS,), jnp.float32),
        ],
        compiler_params=pltpu.CompilerParams(use_tc_tiling_on_sc=False, needs_layout_passes=False),
    )
    def k(dst_hbm, cnt0_hbm, cnt1_hbm, dst_v, ones_v, zc_v, cnt_sp):
        c = lax.axis_index("c")
        s = lax.axis_index("s")
        wid = c * 16 + s

        def fill(i, _):
            ones_v[pl.ds(i * 16, 16)] = jnp.full((16,), 1.0, jnp.float32)
            return 0
        lax.fori_loop(0, WIN // 16, fill, 0)

        def zz(i, _):
            zc_v[pl.ds(i * 16, 16)] = jnp.zeros((16,), jnp.float32)
            return 0
        lax.fori_loop(0, 6432 // 16, zz, 0)
        pltpu.sync_copy(zc_v, cnt_sp.at[pl.ds(s * 6432, 6432)])
        plsc.subcore_barrier()

        def win(w, _):
            base = wid * (E_PAD // 32) + w * WIN
            pltpu.sync_copy(dst_hbm.at[pl.ds(base, WIN)], dst_v)
            pltpu.sync_copy(ones_v, cnt_sp.at[dst_v], add=True)
            return 0
        lax.fori_loop(0, (E_PAD // 32) // WIN, win, 0)
        plsc.subcore_barrier()

        pltpu.sync_copy(cnt_sp.at[pl.ds(s * 6432, 6432)], zc_v)

        @pl.when(c == 0)
        def _():
            pltpu.sync_copy(zc_v, cnt0_hbm.at[pl.ds(s * 6432, 6432)])

        @pl.when(c == 1)
        def _():
            pltpu.sync_copy(zc_v, cnt1_hbm.at[pl.ds(s * 6432, 6432)])

    return k(dst_p)


def _conv(h, src_p, dst_p):
    @functools.partial(
        pl.kernel,
        out_type=jax.ShapeDtypeStruct((8 * GROUP, HID), jnp.float32),
        mesh=_MESH,
        scratch_types=[
            pltpu.VMEM((WIN,), jnp.int32),
            pltpu.VMEM((WIN,), jnp.int32),
            pltpu.VMEM((WIN,), jnp.int32),
            pltpu.VMEM((WIN,), jnp.int32),
            pltpu.VMEM((WIN, HID), jnp.float32),
            pltpu.VMEM((832, HID), jnp.float32),
            pltpu.VMEM_SHARED((AGG_ROWS, HID), jnp.float32),
            pltpu.SemaphoreType.DMA,
        ],
        compiler_params=pltpu.CompilerParams(use_tc_tiling_on_sc=False, needs_layout_passes=False),
    )
    def k(h_hbm, src_hbm, dst_hbm, agg_hbm,
          src_v, dst_v, psrc_v, pldst_v, rows_v, out_v, agg_sp, sem):
        c = lax.axis_index("c")
        s = lax.axis_index("s")
        lanes = lax.iota(jnp.int32, 16)
        pad_src = s * 128 + lanes * 8
        pad_ldst = GROUP + s * 32 + lanes

        def flush():
            pltpu.async_copy(h_hbm.at[psrc_v], rows_v, sem).wait()
            pltpu.sync_copy(rows_v, agg_sp.at[pldst_v], add=True)

        def pad_to(cnt, hi):
            m = lanes < hi - cnt
            idx = jnp.minimum(cnt + lanes, WIN - 1)
            plsc.store_scatter(psrc_v, [idx], pad_src, mask=m)
            plsc.store_scatter(pldst_v, [idx], pad_ldst, mask=m)

        for p in range(NPASS):
            qbase = (2 * p + c) * GROUP

            def z(i, _):
                out_v[i, pl.ds(0, 16)] = jnp.zeros((16,), jnp.float32)
                out_v[i, pl.ds(16, 16)] = jnp.zeros((16,), jnp.float32)
                return 0
            lax.fori_loop(0, 832, z, 0)
            pltpu.sync_copy(out_v, agg_sp.at[pl.ds(s * 832, 832)])
            plsc.subcore_barrier()

            def win(w, cnt):
                base = s * (E_PAD // 16) + w * WIN
                pltpu.sync_copy(src_hbm.at[pl.ds(base, WIN)], src_v)
                pltpu.sync_copy(dst_hbm.at[pl.ds(base, WIN)], dst_v)

                def grp(j, cnt):
                    d = dst_v[pl.ds(j * 16, 16)] - qbase
                    ok = jnp.logical_and(d >= 0, d < GROUP)
                    gc = plsc.all_reduce_population_count(ok)[0]
                    full = cnt > WIN - 16

                    @pl.when(full)
                    def _():
                        pad_to(cnt, jnp.int32(WIN))
                        flush()

                    cnt = jnp.where(full, 0, cnt)
                    sv = src_v[pl.ds(j * 16, 16)]
                    plsc.store_compressed(psrc_v.at[pl.ds(cnt, 16)], sv, mask=ok)
                    plsc.store_compressed(pldst_v.at[pl.ds(cnt, 16)], d, mask=ok)
                    return cnt + gc
                return lax.fori_loop(0, WIN // 16, grp, cnt)
            cnt = lax.fori_loop(0, (E_PAD // 16) // WIN, win, jnp.int32(0))

            def tail(i, cnt):
                pad_to(cnt, jnp.int32(WIN))
                return jnp.minimum(cnt + 16, WIN)
            lax.fori_loop(0, WIN // 16, tail, cnt)
            flush()
            plsc.subcore_barrier()

            ro = s * 800
            ob = (2 * p + c) * GROUP
            pltpu.sync_copy(agg_sp.at[pl.ds(ro, 800)],
                            out_v.at[pl.ds(0, 800)])
            pltpu.sync_copy(out_v.at[pl.ds(0, 800)],
                            agg_hbm.at[pl.ds(ob + ro, 800)])

            plsc.subcore_barrier()

    return k(h, src_p, dst_p)


def _mm_relu_body(x_ref, w_ref, b_ref, o_ref):
    o_ref[...] = jax.nn.relu(
        jnp.dot(x_ref[...], w_ref[...], preferred_element_type=jnp.float32)
        + b_ref[...]
    )


def _mm_relu(x, w, b):
    m, kk = x.shape
    n = w.shape[1]
    return pl.pallas_call(
        _mm_relu_body,
        grid=(m // ROW_BLK,),
        in_specs=[
            pl.BlockSpec((ROW_BLK, kk), lambda i: (i, 0)),
            pl.BlockSpec((kk, n), lambda i: (0, 0)),
            pl.BlockSpec((1, n), lambda i: (0, 0)),
        ],
        out_specs=pl.BlockSpec((ROW_BLK, n), lambda i: (i, 0)),
        out_shape=jax.ShapeDtypeStruct((m, n), jnp.float32),
    )(x, w, b.reshape(1, n))


def _combine_body(agg_ref, c0_ref, c1_ref, h_ref, wl_ref, wr_ref, b_ref,
                  o_ref):
    cnt = jnp.maximum(c0_ref[...] + c1_ref[...], 1.0)
    o_ref[...] = jax.nn.relu(
        jnp.dot(agg_ref[...] / cnt, wl_ref[...],
                preferred_element_type=jnp.float32)
        + jnp.dot(h_ref[...], wr_ref[...], preferred_element_type=jnp.float32)
        + b_ref[...]
    )


def _combine2_body(agg_ref, c0_ref, c1_ref, h_ref, wl_ref, wr_ref, b_ref,
                   w2_ref, b2_ref, o_ref):
    cnt = jnp.maximum(c0_ref[...] + c1_ref[...], 1.0)
    t = jax.nn.relu(
        jnp.dot(agg_ref[...] / cnt, wl_ref[...],
                preferred_element_type=jnp.float32)
        + jnp.dot(h_ref[...], wr_ref[...], preferred_element_type=jnp.float32)
        + b_ref[...]
    )
    o_ref[...] = jax.nn.relu(
        jnp.dot(t, w2_ref[...], preferred_element_type=jnp.float32)
        + b2_ref[...]
    )


def _combine(agg, c0, c1, h, wl, wr, b, w2=None, b2=None):
    m, n = h.shape
    fused = w2 is not None
    body = _combine2_body if fused else _combine_body
    specs = [
        pl.BlockSpec((ROW_BLK, n), lambda i: (i, 0)),
        pl.BlockSpec((ROW_BLK, 1), lambda i: (i, 0)),
        pl.BlockSpec((ROW_BLK, 1), lambda i: (i, 0)),
        pl.BlockSpec((ROW_BLK, n), lambda i: (i, 0)),
        pl.BlockSpec((n, n), lambda i: (0, 0)),
        pl.BlockSpec((n, n), lambda i: (0, 0)),
        pl.BlockSpec((1, n), lambda i: (0, 0)),
    ]
    args = [agg, c0, c1, h, wl, wr, b.reshape(1, n)]
    if fused:
        specs += [
            pl.BlockSpec((n, n), lambda i: (0, 0)),
            pl.BlockSpec((1, n), lambda i: (0, 0)),
        ]
        args += [w2, b2.reshape(1, n)]
    return pl.pallas_call(
        body,
        grid=(m // ROW_BLK,),
        in_specs=specs,
        out_specs=pl.BlockSpec((ROW_BLK, n), lambda i: (i, 0)),
        out_shape=jax.ShapeDtypeStruct((m, n), jnp.float32),
    )(*args)


def kernel(x, edge_index, W1, b1, W2, b2, Wl1, Wr1, bc1, Wl2, Wr2, bc2):
    src = edge_index[0].astype(jnp.int32)
    dst = edge_index[1].astype(jnp.int32)

    src_p = jnp.concatenate([src, jnp.zeros((PAD_N,), jnp.int32)])
    dst_p = jnp.concatenate(
        [dst, 8 * GROUP + (jnp.arange(PAD_N, dtype=jnp.int32) & 255)])

    cnt0, cnt1 = _prep(dst_p)
    c0 = cnt0[:N_NODES].reshape(N_NODES, 1)
    c1 = cnt1[:N_NODES].reshape(N_NODES, 1)

    h0 = _mm_relu(x, W1, b1)
    agg1 = _conv(h0, src_p, dst_p)[:N_NODES]
    h2 = _combine(agg1, c0, c1, h0, Wl1, Wr1, bc1, W2, b2)
    agg2 = _conv(h2, src_p, dst_p)[:N_NODES]
    return _combine(agg2, c0, c1, h2, Wl2, Wr2, bc2)
